# baseline (speedup 1.0000x reference)
import functools

import jax
import jax.numpy as jnp
from jax import lax
from jax.experimental import pallas as pl
from jax.experimental.pallas import tpu as pltpu
from jax.experimental.pallas import tpu_sc as plsc

TOKEN_NUMS = 100000
PAD_TOKEN = TOKEN_NUMS + 1
EMBED_DIM = 128
BZ, SL, T = 1024, 50, 20
HID = 256

NW = 32
SEG = BZ * T
SW = 64
NSEG_W = SEG // NW
CHS = 80

NID = BZ * SL + BZ
IDS_W = NID // NW
CH = 272

@functools.cache
def _build_sc_gather():
    mesh = plsc.VectorSubcoreMesh(core_axis_name="c", subcore_axis_name="s")
    return functools.partial(
        pl.kernel,
        mesh=mesh,
        out_type=jax.ShapeDtypeStruct((NID, EMBED_DIM), jnp.float32),
        scratch_types=[
            pltpu.VMEM((IDS_W,), jnp.int32),
            pltpu.VMEM((CH, EMBED_DIM), jnp.float32),
            pltpu.SemaphoreType.DMA,
        ],
    )(_sc_gather_body)


def _sc_gather_body(table_hbm, ids_hbm, out_hbm, idx_v, rows_v, sem):
    wid = lax.axis_index("s") * 2 + lax.axis_index("c")
    base = wid * IDS_W
    pltpu.sync_copy(ids_hbm.at[pl.ds(base, IDS_W)], idx_v)
    for c in range(IDS_W // CH):
        pltpu.async_copy(
            table_hbm.at[idx_v.at[pl.ds(c * CH, CH)]], rows_v, sem
        ).wait()
        pltpu.sync_copy(rows_v, out_hbm.at[pl.ds(base + c * CH, CH)])


PAIRW = SW
NBUF = 6


@functools.cache
def _build_sc_agent_sum():
    mesh = plsc.VectorSubcoreMesh(core_axis_name="c", subcore_axis_name="s")
    return functools.partial(
        pl.kernel,
        mesh=mesh,
        out_type=(jax.ShapeDtypeStruct((SEG, EMBED_DIM), jnp.float32),
                  jax.ShapeDtypeStruct((SEG, 16), jnp.float32)),
        scratch_types=[
            pltpu.VMEM((CHS, SW), jnp.float32),
            pltpu.VMEM((CHS, PAIRW), jnp.int32),
            pltpu.VMEM((NBUF, PAIRW, EMBED_DIM), jnp.float32),
            pltpu.VMEM((CHS, EMBED_DIM), jnp.float32),
            pltpu.VMEM((CHS, 16), jnp.float32),
            pltpu.SemaphoreType.DMA((NBUF,)),
        ],
    )(_sc_agent_sum_body)


def _sc_agent_sum_body(table_hbm, tok_hbm, out_hbm, cnt_hbm, tok_v, idx_v,
                       rows_v, outb_v, outc_v, sems):
    wid = lax.axis_index("s") * 2 + lax.axis_index("c")
    base = wid * NSEG_W

    def start(pr, b):
        pltpu.make_async_copy(
            table_hbm.at[idx_v.at[pr]], rows_v.at[b], sems.at[b]
        ).start()

    def wait(pr, b):
        pltpu.make_async_copy(
            table_hbm.at[idx_v.at[pr]], rows_v.at[b], sems.at[b]
        ).wait()

    def sum_seg(b, li):
        def body4(j, accs):
            r0 = 4 * j
            new = []
            for c in range(8):
                sl = pl.ds(c * 16, 16)
                v0 = rows_v[b, r0, sl]
                v1 = rows_v[b, r0 + 1, sl]
                v2 = rows_v[b, r0 + 2, sl]
                v3 = rows_v[b, r0 + 3, sl]
                new.append(accs[c] + ((v0 + v1) + (v2 + v3)))
            return tuple(new)

        accs = lax.fori_loop(
            0, SW // 4, body4,
            tuple(jnp.zeros((16,), jnp.float32) for _ in range(8)),
        )
        for c in range(8):
            outb_v[li, pl.ds(c * 16, 16)] = accs[c]

    for ch in range(NSEG_W // CHS):
        cb = ch * CHS
        pltpu.sync_copy(tok_hbm.at[pl.ds(base + cb, CHS)], tok_v)

        def idx_body(i, _):
            cntv = None
            for c0 in (0, 16, 32, 48):
                t = tok_v[i, pl.ds(c0, 16)]
                msk = t != -1.0
                ti = jnp.clip(t.astype(jnp.int32), 0, TOKEN_NUMS + 2)
                idx_v[i, pl.ds(c0, 16)] = jnp.where(msk, ti, PAD_TOKEN)
                ones = jnp.where(msk, 1.0, 0.0)
                cntv = ones if cntv is None else cntv + ones
            outc_v[i, :] = cntv
            return 0

        lax.fori_loop(0, CHS, idx_body, 0)

        for b in range(NBUF):
            start(b, b)

        def segloop(p, _):
            b = lax.rem(p, NBUF)
            wait(p, b)

            @pl.when(p + NBUF < CHS)
            def _():
                start(p + NBUF, b)

            sum_seg(b, p)
            return 0

        lax.fori_loop(0, CHS, segloop, 0)
        pltpu.sync_copy(outb_v, out_hbm.at[pl.ds(base + cb, CHS)])
        pltpu.sync_copy(outc_v, cnt_hbm.at[pl.ds(base + cb, CHS)])


def _tc_self(emb_cat, ego, w1a, w1b, b1, w2, b2):
    def body(emb_ref, ego_ref, w1a_ref, w1b_ref, b1_ref, w2_ref, b2_ref,
             out_ref):
        h = jnp.maximum(
            jnp.dot(emb_ref[:], w1a_ref[:], preferred_element_type=jnp.float32)
            + jnp.dot(ego_ref[:], w1b_ref[:],
                      preferred_element_type=jnp.float32)
            + b1_ref[:], 0.0)
        out_ref[:] = (
            jnp.dot(h, w2_ref[:], preferred_element_type=jnp.float32)
            + b2_ref[:])

    return pl.pallas_call(
        body,
        grid=(SL,),
        in_specs=[
            pl.BlockSpec((BZ, EMBED_DIM), lambda i: (i, 0)),
            pl.BlockSpec((BZ, 3), lambda i: (0, 0)),
            pl.BlockSpec((EMBED_DIM, HID), lambda i: (0, 0)),
            pl.BlockSpec((3, HID), lambda i: (0, 0)),
            pl.BlockSpec((1, HID), lambda i: (0, 0)),
            pl.BlockSpec((HID, EMBED_DIM), lambda i: (0, 0)),
            pl.BlockSpec((1, EMBED_DIM), lambda i: (0, 0)),
        ],
        out_specs=pl.BlockSpec((BZ, EMBED_DIM), lambda i: (i, 0)),
        out_shape=jax.ShapeDtypeStruct((BZ * SL, EMBED_DIM), jnp.float32),
        compiler_params=pltpu.CompilerParams(
            dimension_semantics=("parallel",)),
    )(emb_cat, ego, w1a, w1b, b1, w2, b2)


BB = 16
SEGB = BB * T
EPR = 16
RPB = SEGB * SW // EPR


def _tc_env(af2, asum, cnt16, emb_cat, pad_row, w_bd, p_tok, w1p, w1g, b1,
            w2, b2):
    def body(af_ref, asum_ref, cnt_ref, g_ref, pad_ref, wbd_ref, ptok_ref,
             w1p_ref, w1g_ref, b1_ref, w2_ref, b2_ref, out_ref):
        af2_blk = af_ref[:]
        tokb = jnp.dot(af2_blk, ptok_ref[:],
                       preferred_element_type=jnp.float32)
        m = (tokb != -1.0).astype(jnp.float32)
        ch = lax.broadcasted_iota(jnp.int32, (1, 128), 1) % 8
        keep = (ch != 0).astype(jnp.float32)
        sel = 1.0 - keep
        af_in = m * (af2_blk * keep + sel)
        z = jnp.dot(af_in, wbd_ref[:],
                    preferred_element_type=jnp.float32)
        f = jnp.maximum(z, 0.0)
        gsum = f[:, :EMBED_DIM]
        for g in range(1, EPR):
            gsum = gsum + f[:, g * EMBED_DIM:(g + 1) * EMBED_DIM]
        fsum = jnp.sum(gsum.reshape(SEGB, SW // EPR, EMBED_DIM), axis=1)
        cnt = jnp.sum(cnt_ref[:], axis=1, keepdims=True)
        esum = asum_ref[:] - (float(SW) - cnt) * pad_ref[:]
        pooled = (esum + fsum) / jnp.clip(cnt, 1.0, None)
        g2 = jnp.dot(g_ref[:], w1g_ref[:],
                     preferred_element_type=jnp.float32)
        g2b = jnp.broadcast_to(g2[:, None, :], (BB, T, HID)).reshape(SEGB, HID)
        hb = jnp.maximum(
            jnp.dot(pooled, w1p_ref[:], preferred_element_type=jnp.float32)
            + g2b + b1_ref[:], 0.0)
        out_ref[:] = (
            jnp.dot(hb, w2_ref[:], preferred_element_type=jnp.float32)
            + b2_ref[:])

    return pl.pallas_call(
        body,
        grid=(BZ // BB,),
        in_specs=[
            pl.BlockSpec((RPB, 128), lambda i: (i, 0)),
            pl.BlockSpec((SEGB, EMBED_DIM), lambda i: (i, 0)),
            pl.BlockSpec((SEGB, 16), lambda i: (i, 0)),
            pl.BlockSpec((BB, EMBED_DIM), lambda i: (BZ * SL // BB + i, 0)),
            pl.BlockSpec((1, EMBED_DIM), lambda i: (0, 0)),
            pl.BlockSpec((128, EPR * EMBED_DIM), lambda i: (0, 0)),
            pl.BlockSpec((128, 128), lambda i: (0, 0)),
            pl.BlockSpec((EMBED_DIM, HID), lambda i: (0, 0)),
            pl.BlockSpec((EMBED_DIM, HID), lambda i: (0, 0)),
            pl.BlockSpec((1, HID), lambda i: (0, 0)),
            pl.BlockSpec((HID, EMBED_DIM), lambda i: (0, 0)),
            pl.BlockSpec((1, EMBED_DIM), lambda i: (0, 0)),
        ],
        out_specs=pl.BlockSpec((SEGB, EMBED_DIM), lambda i: (i, 0)),
        out_shape=jax.ShapeDtypeStruct((SEG, EMBED_DIM), jnp.float32),
        compiler_params=pltpu.CompilerParams(
            dimension_semantics=("parallel",)),
    )(af2, asum, cnt16, emb_cat, pad_row, w_bd, p_tok, w1p, w1g, b1, w2, b2)


def kernel(input_ids, ego_info, agent_info, goal, token_table,
           W_s1, b_s1, W_s2, b_s2, W_f, b_f, W_b1, b_b1, W_b2, b_b2):
    ids_cat = jnp.concatenate(
        [input_ids.reshape(-1), goal]).astype(jnp.int32)
    emb_cat = _build_sc_gather()(token_table, ids_cat)

    tok = agent_info[..., 0].reshape(SEG, SL)
    tok_p = jnp.concatenate(
        [tok, jnp.full((SEG, SW - SL), -1.0, tok.dtype)], axis=1)
    asum, cnt16 = _build_sc_agent_sum()(token_table, tok_p)

    nch = 1 + W_f.shape[0]
    af_pad = jnp.pad(agent_info, ((0, 0), (0, 0), (0, SW - SL), (0, 0)),
                     constant_values=-1.0)
    af2 = af_pad.reshape(SEG * SW * nch // 128, 128)
    wf_blk = jnp.concatenate([b_f[None], W_f], axis=0)
    w_bd = jnp.kron(jnp.eye(EPR, dtype=W_f.dtype), wf_blk)
    p_blk = jnp.zeros((nch, nch), W_f.dtype).at[0, :].set(1.0)
    p_tok = jnp.kron(jnp.eye(EPR, dtype=W_f.dtype), p_blk)
    pad_row = token_table[PAD_TOKEN][None]

    self_flat = _tc_self(
        emb_cat, ego_info, W_s1[:EMBED_DIM], W_s1[EMBED_DIM:],
        b_s1[None], W_s2, b_s2[None])
    env_flat = _tc_env(
        af2, asum, cnt16, emb_cat, pad_row, w_bd, p_tok,
        W_b1[:EMBED_DIM], W_b1[EMBED_DIM:],
        b_b1[None], W_b2, b_b2[None])
    return (self_flat.reshape(BZ, SL, EMBED_DIM),
            env_flat.reshape(BZ, T, EMBED_DIM))


# speedup vs baseline: 5.5492x; 5.5492x over previous
import functools

import jax
import jax.numpy as jnp
from jax import lax
from jax.experimental import pallas as pl
from jax.experimental.pallas import tpu as pltpu
from jax.experimental.pallas import tpu_sc as plsc

TOKEN_NUMS = 100000
PAD_TOKEN = TOKEN_NUMS + 1
EMBED_DIM = 128
BZ, SL, T = 1024, 50, 20
HID = 256

NW = 32
SEG = BZ * T
SW = 64
NSEG_W = SEG // NW
CHS = 80

ZROW0 = TOKEN_NUMS + 3

NID = BZ * SL + BZ
IDS_W = NID // NW
CH = 272

@functools.cache
def _build_sc_gather():
    mesh = plsc.VectorSubcoreMesh(core_axis_name="c", subcore_axis_name="s")
    return functools.partial(
        pl.kernel,
        mesh=mesh,
        out_type=jax.ShapeDtypeStruct((NID, EMBED_DIM), jnp.float32),
        scratch_types=[
            pltpu.VMEM((IDS_W,), jnp.int32),
            pltpu.VMEM((CH, EMBED_DIM), jnp.float32),
            pltpu.SemaphoreType.DMA,
        ],
    )(_sc_gather_body)


def _sc_gather_body(table_hbm, ids_hbm, out_hbm, idx_v, rows_v, sem):
    wid = lax.axis_index("s") * 2 + lax.axis_index("c")
    base = wid * IDS_W
    pltpu.sync_copy(ids_hbm.at[pl.ds(base, IDS_W)], idx_v)
    for c in range(IDS_W // CH):
        pltpu.async_copy(
            table_hbm.at[idx_v.at[pl.ds(c * CH, CH)]], rows_v, sem
        ).wait()
        pltpu.sync_copy(rows_v, out_hbm.at[pl.ds(base + c * CH, CH)])


PAIRW = SW
NBUF = 6


@functools.cache
def _build_sc_agent_sum():
    mesh = plsc.VectorSubcoreMesh(core_axis_name="c", subcore_axis_name="s")
    return functools.partial(
        pl.kernel,
        mesh=mesh,
        out_type=(jax.ShapeDtypeStruct((SEG, EMBED_DIM), jnp.float32),
                  jax.ShapeDtypeStruct((SEG, 16), jnp.float32)),
        scratch_types=[
            pltpu.VMEM((CHS, SW), jnp.float32),
            pltpu.VMEM((CHS, PAIRW), jnp.int32),
            pltpu.VMEM((NBUF, PAIRW, EMBED_DIM), jnp.float32),
            pltpu.VMEM((CHS, EMBED_DIM), jnp.float32),
            pltpu.VMEM((CHS, 16), jnp.float32),
            pltpu.SemaphoreType.DMA((NBUF,)),
        ],
    )(_sc_agent_sum_body)


def _sc_agent_sum_body(table_hbm, tok_hbm, out_hbm, cnt_hbm, tok_v, idx_v,
                       rows_v, outb_v, outc_v, sems):
    wid = lax.axis_index("s") * 2 + lax.axis_index("c")
    base = wid * NSEG_W

    def start(pr, b):
        pltpu.make_async_copy(
            table_hbm.at[idx_v.at[pr]], rows_v.at[b], sems.at[b]
        ).start()

    def wait(pr, b):
        pltpu.make_async_copy(
            table_hbm.at[idx_v.at[pr]], rows_v.at[b], sems.at[b]
        ).wait()

    def sum_seg(b, li):
        def body4(j, accs):
            r0 = 4 * j
            new = []
            for c in range(8):
                sl = pl.ds(c * 16, 16)
                v0 = rows_v[b, r0, sl]
                v1 = rows_v[b, r0 + 1, sl]
                v2 = rows_v[b, r0 + 2, sl]
                v3 = rows_v[b, r0 + 3, sl]
                new.append(accs[c] + ((v0 + v1) + (v2 + v3)))
            return tuple(new)

        accs = lax.fori_loop(
            0, SW // 4, body4,
            tuple(jnp.zeros((16,), jnp.float32) for _ in range(8)),
        )
        for c in range(8):
            outb_v[li, pl.ds(c * 16, 16)] = accs[c]

    for ch in range(NSEG_W // CHS):
        cb = ch * CHS
        pltpu.sync_copy(tok_hbm.at[pl.ds(base + cb, CHS)], tok_v)

        def idx_body(i, _):
            cntv = None
            for c0 in (0, 16, 32, 48):
                t = tok_v[i, pl.ds(c0, 16)]
                ge0 = t >= 0.0
                ti = jnp.clip(t.astype(jnp.int32), 0, TOKEN_NUMS + 2)
                pad_idx = ZROW0 - t.astype(jnp.int32) - 2
                idx_v[i, pl.ds(c0, 16)] = jnp.where(
                    ge0, ti, jnp.where(t == -1.0, PAD_TOKEN, pad_idx))
                ones = jnp.where(ge0, 1.0, 0.0)
                cntv = ones if cntv is None else cntv + ones
            outc_v[i, :] = cntv
            return 0

        lax.fori_loop(0, CHS, idx_body, 0)

        for b in range(NBUF):
            start(b, b)

        def segloop(p, _):
            b = lax.rem(p, NBUF)
            wait(p, b)

            @pl.when(p + NBUF < CHS)
            def _():
                start(p + NBUF, b)

            sum_seg(b, p)
            return 0

        lax.fori_loop(0, CHS, segloop, 0)
        pltpu.sync_copy(outb_v, out_hbm.at[pl.ds(base + cb, CHS)])
        pltpu.sync_copy(outc_v, cnt_hbm.at[pl.ds(base + cb, CHS)])


def _tc_self(emb_cat, ego, w1a, w1b, b1, w2, b2):
    def body(emb_ref, ego_ref, w1a_ref, w1b_ref, b1_ref, w2_ref, b2_ref,
             out_ref):
        h = jnp.maximum(
            jnp.dot(emb_ref[:], w1a_ref[:], preferred_element_type=jnp.float32)
            + jnp.dot(ego_ref[:], w1b_ref[:],
                      preferred_element_type=jnp.float32)
            + b1_ref[:], 0.0)
        out_ref[:] = (
            jnp.dot(h, w2_ref[:], preferred_element_type=jnp.float32)
            + b2_ref[:])

    return pl.pallas_call(
        body,
        grid=(SL,),
        in_specs=[
            pl.BlockSpec((BZ, EMBED_DIM), lambda i: (i, 0)),
            pl.BlockSpec((BZ, 3), lambda i: (0, 0)),
            pl.BlockSpec((EMBED_DIM, HID), lambda i: (0, 0)),
            pl.BlockSpec((3, HID), lambda i: (0, 0)),
            pl.BlockSpec((1, HID), lambda i: (0, 0)),
            pl.BlockSpec((HID, EMBED_DIM), lambda i: (0, 0)),
            pl.BlockSpec((1, EMBED_DIM), lambda i: (0, 0)),
        ],
        out_specs=pl.BlockSpec((BZ, EMBED_DIM), lambda i: (i, 0)),
        out_shape=jax.ShapeDtypeStruct((BZ * SL, EMBED_DIM), jnp.float32),
        compiler_params=pltpu.CompilerParams(
            dimension_semantics=("parallel",)),
    )(emb_cat, ego, w1a, w1b, b1, w2, b2)


BB = 16
SEGB = BB * T
EPR = 16
RPB = SEGB * SW // EPR


def _tc_env(af2, asum, cnt16, emb_cat, pad_row, w_bd, p_tok, w1p, w1g, b1,
            w2, b2):
    def body(af_ref, asum_ref, cnt_ref, g_ref, pad_ref, wbd_ref, ptok_ref,
             w1p_ref, w1g_ref, b1_ref, w2_ref, b2_ref, out_ref):
        af2_blk = af_ref[:]
        tokb = jnp.dot(af2_blk, ptok_ref[:],
                       preferred_element_type=jnp.float32)
        m = (tokb != -1.0).astype(jnp.float32)
        ch = lax.broadcasted_iota(jnp.int32, (1, 128), 1) % 8
        keep = (ch != 0).astype(jnp.float32)
        sel = 1.0 - keep
        af_in = m * (af2_blk * keep + sel)
        z = jnp.dot(af_in, wbd_ref[:],
                    preferred_element_type=jnp.float32)
        f = jnp.maximum(z, 0.0)
        gsum = f[:, :EMBED_DIM]
        for g in range(1, EPR):
            gsum = gsum + f[:, g * EMBED_DIM:(g + 1) * EMBED_DIM]
        fsum = jnp.sum(gsum.reshape(SEGB, SW // EPR, EMBED_DIM), axis=1)
        cnt = jnp.sum(cnt_ref[:], axis=1, keepdims=True)
        esum = asum_ref[:] - (float(SL) - cnt) * pad_ref[:]
        pooled = (esum + fsum) / jnp.clip(cnt, 1.0, None)
        g2 = jnp.dot(g_ref[:], w1g_ref[:],
                     preferred_element_type=jnp.float32)
        g2b = jnp.broadcast_to(g2[:, None, :], (BB, T, HID)).reshape(SEGB, HID)
        hb = jnp.maximum(
            jnp.dot(pooled, w1p_ref[:], preferred_element_type=jnp.float32)
            + g2b + b1_ref[:], 0.0)
        out_ref[:] = (
            jnp.dot(hb, w2_ref[:], preferred_element_type=jnp.float32)
            + b2_ref[:])

    return pl.pallas_call(
        body,
        grid=(BZ // BB,),
        in_specs=[
            pl.BlockSpec((RPB, 128), lambda i: (i, 0)),
            pl.BlockSpec((SEGB, EMBED_DIM), lambda i: (i, 0)),
            pl.BlockSpec((SEGB, 16), lambda i: (i, 0)),
            pl.BlockSpec((BB, EMBED_DIM), lambda i: (BZ * SL // BB + i, 0)),
            pl.BlockSpec((1, EMBED_DIM), lambda i: (0, 0)),
            pl.BlockSpec((128, EPR * EMBED_DIM), lambda i: (0, 0)),
            pl.BlockSpec((128, 128), lambda i: (0, 0)),
            pl.BlockSpec((EMBED_DIM, HID), lambda i: (0, 0)),
            pl.BlockSpec((EMBED_DIM, HID), lambda i: (0, 0)),
            pl.BlockSpec((1, HID), lambda i: (0, 0)),
            pl.BlockSpec((HID, EMBED_DIM), lambda i: (0, 0)),
            pl.BlockSpec((1, EMBED_DIM), lambda i: (0, 0)),
        ],
        out_specs=pl.BlockSpec((SEGB, EMBED_DIM), lambda i: (i, 0)),
        out_shape=jax.ShapeDtypeStruct((SEG, EMBED_DIM), jnp.float32),
        compiler_params=pltpu.CompilerParams(
            dimension_semantics=("parallel",)),
    )(af2, asum, cnt16, emb_cat, pad_row, w_bd, p_tok, w1p, w1g, b1, w2, b2)


def kernel(input_ids, ego_info, agent_info, goal, token_table,
           W_s1, b_s1, W_s2, b_s2, W_f, b_f, W_b1, b_b1, W_b2, b_b2):
    ids_cat = jnp.concatenate(
        [input_ids.reshape(-1), goal]).astype(jnp.int32)
    emb_cat = _build_sc_gather()(token_table, ids_cat)

    tok = agent_info[..., 0].reshape(SEG, SL)
    pad_codes = -(2.0 + jnp.arange(SW - SL, dtype=tok.dtype))
    tok_p = jnp.concatenate(
        [tok, jnp.broadcast_to(pad_codes[None], (SEG, SW - SL))], axis=1)
    table_aug = jnp.concatenate(
        [token_table, jnp.zeros((16, EMBED_DIM), token_table.dtype)], axis=0)
    asum, cnt16 = _build_sc_agent_sum()(table_aug, tok_p)

    nch = 1 + W_f.shape[0]
    af_pad = jnp.pad(agent_info, ((0, 0), (0, 0), (0, SW - SL), (0, 0)),
                     constant_values=-1.0)
    af2 = af_pad.reshape(SEG * SW * nch // 128, 128)
    wf_blk = jnp.concatenate([b_f[None], W_f], axis=0)
    w_bd = jnp.kron(jnp.eye(EPR, dtype=W_f.dtype), wf_blk)
    p_blk = jnp.zeros((nch, nch), W_f.dtype).at[0, :].set(1.0)
    p_tok = jnp.kron(jnp.eye(EPR, dtype=W_f.dtype), p_blk)
    pad_row = token_table[PAD_TOKEN][None]

    self_flat = _tc_self(
        emb_cat, ego_info, W_s1[:EMBED_DIM], W_s1[EMBED_DIM:],
        b_s1[None], W_s2, b_s2[None])
    env_flat = _tc_env(
        af2, asum, cnt16, emb_cat, pad_row, w_bd, p_tok,
        W_b1[:EMBED_DIM], W_b1[EMBED_DIM:],
        b_b1[None], W_b2, b_b2[None])
    return (self_flat.reshape(BZ, SL, EMBED_DIM),
            env_flat.reshape(BZ, T, EMBED_DIM))
